# baseline (speedup 1.0000x reference)
import functools

import numpy as np
import jax
import jax.numpy as jnp
from jax import lax
from jax.experimental import pallas as pl
from jax.experimental.pallas import tpu as pltpu
from jax.experimental.pallas import tpu_sc as plsc

_NUM_USERS = 50000
_NUM_ITEMS = 50000
_N_NODES = _NUM_USERS + _NUM_ITEMS
_DROP = 0.1
_E = 6400000

_NC = 2
_NS = 16
_NW = _NC * _NS
_C = 6400
_NCHUNK = _E // _C
_NBUF = 3
_MAXJ = (_NCHUNK + _NW - 1) // _NW
_MAXJ_PAD = -(-_MAXJ // _NBUF) * _NBUF
_VPC = _C // 16
_NWORDS = _N_NODES // 32
_NWORDS_PAD = 3200

_N_DROP_U = int(_NUM_USERS * _DROP)
_N_DROP_I = int(_NUM_ITEMS * _DROP)
_KEEP_CACHE: list = []


def _keep_words() -> jax.Array:
    if _KEEP_CACHE:
        return jnp.asarray(_KEEP_CACHE[0])
    try:
        with jax.ensure_compile_time_eval():
            ku, ki = jax.random.split(jax.random.key(42))
            user_perm = np.asarray(jax.random.permutation(ku, _NUM_USERS))
            item_perm = np.asarray(jax.random.permutation(ki, _NUM_ITEMS))
        flag = np.zeros((_N_NODES,), np.bool_)
        flag[user_perm[:_N_DROP_U]] = True
        flag[item_perm[:_N_DROP_I] + _NUM_USERS] = True
        idx = np.arange(_N_NODES)
        words = np.zeros((_NWORDS_PAD,), np.uint32)
        np.bitwise_or.at(
            words, idx >> 5,
            (~flag).astype(np.uint32) << (31 - (idx & 31)).astype(np.uint32))
        packed = words.view(np.int32)
        _KEEP_CACHE.append(packed)
        return jnp.asarray(packed)
    except Exception:
        ku, ki = jax.random.split(jax.random.key(42))
        user_perm = jax.random.permutation(ku, _NUM_USERS)
        item_perm = jax.random.permutation(ki, _NUM_ITEMS)
        flag = jnp.zeros((_N_NODES,), bool)
        flag = flag.at[user_perm[:_N_DROP_U]].set(True)
        flag = flag.at[item_perm[:_N_DROP_I] + _NUM_USERS].set(True)
        lanes = (~flag).reshape(_NWORDS, 32).astype(jnp.uint32)
        words = jnp.sum(
            lanes << (31 - jnp.arange(32, dtype=jnp.uint32))[None, :],
            axis=1, dtype=jnp.uint32)
        words = jnp.concatenate(
            [words, jnp.zeros((_NWORDS_PAD - _NWORDS,), jnp.uint32)])
        return lax.bitcast_convert_type(words, jnp.int32)


_MESH = plsc.VectorSubcoreMesh(core_axis_name="c", subcore_axis_name="s")


@functools.partial(
    pl.kernel,
    out_type=jax.ShapeDtypeStruct((_E,), jnp.float32),
    mesh=_MESH,
    compiler_params=pltpu.CompilerParams(needs_layout_passes=False),
    scratch_types=[
        pltpu.VMEM((_NWORDS_PAD,), jnp.int32),
        pltpu.VMEM((2, _C), jnp.int32),
        pltpu.VMEM((2, _C), jnp.int32),
        pltpu.VMEM((2, _C), jnp.int32),
        pltpu.VMEM((_C,), jnp.float32),
        pltpu.VMEM((_C,), jnp.float32),
        pltpu.VMEM((_C,), jnp.float32),
        pltpu.VMEM((_C,), jnp.float32),
        pltpu.VMEM((_C,), jnp.float32),
        pltpu.VMEM((_C,), jnp.float32),
        pltpu.SemaphoreType.DMA,
        pltpu.SemaphoreType.DMA,
        pltpu.SemaphoreType.DMA,
        pltpu.SemaphoreType.DMA,
        pltpu.SemaphoreType.DMA,
        pltpu.SemaphoreType.DMA,
    ],
)
def _node_dropout_sc(kw_hbm, edge_hbm, vals_hbm, out_hbm,
                     kw_v, e_v0, e_v1, e_v2, v_v0, v_v1, v_v2,
                     o_v0, o_v1, o_v2,
                     isem0, isem1, isem2, osem0, osem1, osem2):
    wid = lax.axis_index("s") * _NC + lax.axis_index("c")
    pltpu.sync_copy(kw_hbm, kw_v)
    e_bufs = (e_v0, e_v1, e_v2)
    v_bufs = (v_v0, v_v1, v_v2)
    o_bufs = (o_v0, o_v1, o_v2)
    isems = (isem0, isem1, isem2)
    osems = (osem0, osem1, osem2)

    def issue_in(j, b):
        @pl.when(wid + j * _NW < _NCHUNK)
        def _():
            base = (wid + j * _NW) * _C
            pltpu.async_copy(edge_hbm.at[:, pl.ds(base, _C)], e_bufs[b],
                             isems[b])
            pltpu.async_copy(vals_hbm.at[pl.ds(base, _C)], v_bufs[b],
                             isems[b])

    def wait_in(j, b):
        base = (wid + j * _NW) * _C
        pltpu.make_async_copy(edge_hbm.at[:, pl.ds(base, _C)], e_bufs[b],
                              isems[b]).wait()
        pltpu.make_async_copy(vals_hbm.at[pl.ds(base, _C)], v_bufs[b],
                              isems[b]).wait()

    def issue_out(j, b):
        base = (wid + j * _NW) * _C
        pltpu.async_copy(o_bufs[b], out_hbm.at[pl.ds(base, _C)], osems[b])

    def wait_out(j, b):
        base = (wid + j * _NW) * _C
        pltpu.make_async_copy(o_bufs[b], out_hbm.at[pl.ds(base, _C)],
                              osems[b]).wait()

    def compute(b):
        e_v, v_v, o_v = e_bufs[b], v_bufs[b], o_bufs[b]

        @plsc.parallel_loop(0, _VPC, unroll=8)
        def _(i):
            sl = pl.ds(i * 16, 16)
            s = e_v[0, sl]
            d = e_v[1, sl]
            ws = plsc.load_gather(kw_v, [lax.shift_right_logical(s, 5)])
            wd = plsc.load_gather(kw_v, [lax.shift_right_logical(d, 5)])
            both = lax.shift_right_arithmetic(
                (ws << (s & 31)) & (wd << (d & 31)), 31)
            o_v[sl] = lax.bitcast_convert_type(
                lax.bitcast_convert_type(v_v[sl], jnp.int32) & both,
                jnp.float32)

    issue_in(0, 0)
    issue_in(1, 1)

    def dstep(jj, carry):
        for b in range(_NBUF):
            j = jj * _NBUF + b

            @pl.when(wid + j * _NW < _NCHUNK)
            def _(j=j, b=b):
                wait_in(j, b)
                issue_in(j + 2, (b + 2) % _NBUF)

                @pl.when(j >= _NBUF)
                def _():
                    wait_out(j - _NBUF, b)

                compute(b)
                issue_out(j, b)

        return carry

    lax.fori_loop(0, _MAXJ_PAD // _NBUF, dstep, 0)

    for b in range(_NBUF):
        wait_out(b, b)


def kernel(edge_index, values):
    kw = _keep_words()
    return _node_dropout_sc(kw, edge_index, values)


# speedup vs baseline: 1.1198x; 1.1198x over previous
import functools

import numpy as np
import jax
import jax.numpy as jnp
from jax import lax
from jax.experimental import pallas as pl
from jax.experimental.pallas import tpu as pltpu
from jax.experimental.pallas import tpu_sc as plsc

_NUM_USERS = 50000
_NUM_ITEMS = 50000
_N_NODES = _NUM_USERS + _NUM_ITEMS
_DROP = 0.1
_E = 6400000

_NC = 2
_NS = 16
_NW = _NC * _NS
_C = 6400
_NCHUNK = _E // _C
_NBUF = 4
_MAXJ = (_NCHUNK + _NW - 1) // _NW
_MAXJ_PAD = -(-_MAXJ // _NBUF) * _NBUF
_VPC = _C // 16
_NWORDS = _N_NODES // 32
_NWORDS_PAD = 3200
_MINCH = _NCHUNK // _NW
_NREM = _NCHUNK % _NW

_N_DROP_U = int(_NUM_USERS * _DROP)
_N_DROP_I = int(_NUM_ITEMS * _DROP)
_KEEP_CACHE: list = []


def _keep_words() -> jax.Array:
    if _KEEP_CACHE:
        return jnp.asarray(_KEEP_CACHE[0])
    try:
        with jax.ensure_compile_time_eval():
            ku, ki = jax.random.split(jax.random.key(42))
            user_perm = np.asarray(jax.random.permutation(ku, _NUM_USERS))
            item_perm = np.asarray(jax.random.permutation(ki, _NUM_ITEMS))
        flag = np.zeros((_N_NODES,), np.bool_)
        flag[user_perm[:_N_DROP_U]] = True
        flag[item_perm[:_N_DROP_I] + _NUM_USERS] = True
        idx = np.arange(_N_NODES)
        words = np.zeros((_NWORDS_PAD,), np.uint32)
        np.bitwise_or.at(
            words, idx >> 5,
            (~flag).astype(np.uint32) << (31 - (idx & 31)).astype(np.uint32))
        packed = words.view(np.int32)
        _KEEP_CACHE.append(packed)
        return jnp.asarray(packed)
    except Exception:
        ku, ki = jax.random.split(jax.random.key(42))
        user_perm = jax.random.permutation(ku, _NUM_USERS)
        item_perm = jax.random.permutation(ki, _NUM_ITEMS)
        flag = jnp.zeros((_N_NODES,), bool)
        flag = flag.at[user_perm[:_N_DROP_U]].set(True)
        flag = flag.at[item_perm[:_N_DROP_I] + _NUM_USERS].set(True)
        lanes = (~flag).reshape(_NWORDS, 32).astype(jnp.uint32)
        words = jnp.sum(
            lanes << (31 - jnp.arange(32, dtype=jnp.uint32))[None, :],
            axis=1, dtype=jnp.uint32)
        words = jnp.concatenate(
            [words, jnp.zeros((_NWORDS_PAD - _NWORDS,), jnp.uint32)])
        return lax.bitcast_convert_type(words, jnp.int32)


_MESH = plsc.VectorSubcoreMesh(core_axis_name="c", subcore_axis_name="s")


@functools.partial(
    pl.kernel,
    out_type=jax.ShapeDtypeStruct((_E,), jnp.float32),
    mesh=_MESH,
    compiler_params=pltpu.CompilerParams(needs_layout_passes=False),
    scratch_types=[
        pltpu.VMEM((_NWORDS_PAD,), jnp.int32),
        pltpu.VMEM((2, _C), jnp.int32),
        pltpu.VMEM((2, _C), jnp.int32),
        pltpu.VMEM((2, _C), jnp.int32),
        pltpu.VMEM((2, _C), jnp.int32),
        pltpu.VMEM((_C,), jnp.float32),
        pltpu.VMEM((_C,), jnp.float32),
        pltpu.VMEM((_C,), jnp.float32),
        pltpu.VMEM((_C,), jnp.float32),
        pltpu.SemaphoreType.DMA,
        pltpu.SemaphoreType.DMA,
        pltpu.SemaphoreType.DMA,
        pltpu.SemaphoreType.DMA,
        pltpu.SemaphoreType.DMA,
        pltpu.SemaphoreType.DMA,
        pltpu.SemaphoreType.DMA,
        pltpu.SemaphoreType.DMA,
    ],
)
def _node_dropout_sc(kw_hbm, edge_hbm, vals_hbm, out_hbm,
                     kw_v, e_v0, e_v1, e_v2, e_v3, vo_0, vo_1, vo_2, vo_3,
                     isem0, isem1, isem2, isem3,
                     osem0, osem1, osem2, osem3):
    wid = lax.axis_index("s") * _NC + lax.axis_index("c")
    pltpu.sync_copy(kw_hbm, kw_v)
    e_bufs = (e_v0, e_v1, e_v2, e_v3)
    vo_bufs = (vo_0, vo_1, vo_2, vo_3)
    isems = (isem0, isem1, isem2, isem3)
    osems = (osem0, osem1, osem2, osem3)
    lane = lax.iota(jnp.int32, 16)
    zeros = jnp.zeros((16,), jnp.float32)

    def issue_in(j, b):
        @pl.when(wid + j * _NW < _NCHUNK)
        def _():
            base = (wid + j * _NW) * _C
            pltpu.async_copy(edge_hbm.at[:, pl.ds(base, _C)], e_bufs[b],
                             isems[b])
            pltpu.async_copy(vals_hbm.at[pl.ds(base, _C)], vo_bufs[b],
                             isems[b])

    def wait_in(j, b):
        base = (wid + j * _NW) * _C
        pltpu.make_async_copy(edge_hbm.at[:, pl.ds(base, _C)], e_bufs[b],
                              isems[b]).wait()
        pltpu.make_async_copy(vals_hbm.at[pl.ds(base, _C)], vo_bufs[b],
                              isems[b]).wait()

    def issue_out(j, b):
        base = (wid + j * _NW) * _C
        pltpu.async_copy(vo_bufs[b], out_hbm.at[pl.ds(base, _C)], osems[b])

    def wait_out(j, b):
        base = (wid + j * _NW) * _C
        pltpu.make_async_copy(vo_bufs[b], out_hbm.at[pl.ds(base, _C)],
                              osems[b]).wait()

    def compute(b):
        e_v, vo_v = e_bufs[b], vo_bufs[b]

        @plsc.parallel_loop(0, _VPC, unroll=16)
        def _(i):
            sl = pl.ds(i * 16, 16)
            s = e_v[0, sl]
            d = e_v[1, sl]
            ws = plsc.load_gather(kw_v, [lax.shift_right_logical(s, 5)])
            wd = plsc.load_gather(kw_v, [lax.shift_right_logical(d, 5)])
            both = (ws << (s & 31)) & (wd << (d & 31))
            plsc.store_scatter(vo_v.at[sl], [lane], zeros, mask=both >= 0)

    issue_in(0, 0)
    issue_in(1, 1)

    def dstep(jj, carry):
        for b in range(_NBUF):
            j = jj * _NBUF + b

            @pl.when(wid + j * _NW < _NCHUNK)
            def _(j=j, b=b):
                wait_in(j, b)
                b2 = (b + 2) % _NBUF

                @pl.when(j >= 2)
                def _():
                    wait_out(j - 2, b2)

                issue_in(j + 2, b2)
                compute(b)
                issue_out(j, b)

        return carry

    lax.fori_loop(0, _MAXJ_PAD // _NBUF, dstep, 0)

    _hi = {(_MINCH - 1) % _NBUF, _MINCH % _NBUF}
    _lo = {(_MINCH - 2) % _NBUF, (_MINCH - 1) % _NBUF}
    for b in range(_NBUF):
        in_lo, in_hi = b in _lo, b in _hi
        if in_lo and in_hi:
            wait_out(b, b)
        elif in_hi:
            @pl.when(wid < _NREM)
            def _(b=b):
                wait_out(b, b)
        elif in_lo:
            @pl.when(wid >= _NREM)
            def _(b=b):
                wait_out(b, b)


def kernel(edge_index, values):
    kw = _keep_words()
    return _node_dropout_sc(kw, edge_index, values)
